# baseline (speedup 1.0000x reference)
import functools

import jax
import jax.numpy as jnp
from jax import lax
from jax.experimental import pallas as pl
from jax.experimental.pallas import tpu as pltpu
from jax.experimental.pallas import tpu_sc as plsc

B, T, D, FF = 2, 2048, 1024, 4096
CAP = T // 2
RC = 256


def _router_body(x_ref, w_ref, perm_ref):
    xb = x_ref[0]
    w = w_ref[...]
    s_col = lax.dot_general(
        xb.astype(jnp.bfloat16), w.astype(jnp.bfloat16),
        (((1,), (0,)), ((), ())),
        preferred_element_type=jnp.float32)

    pieces = []
    for c in range(T // RC):
        ii = lax.broadcasted_iota(jnp.int32, (T, RC), 0)
        jj = lax.broadcasted_iota(jnp.int32, (T, RC), 1)
        sel = ii == (jj + c * RC)
        pieces.append(jnp.max(jnp.where(sel, s_col, -jnp.inf), axis=0,
                              keepdims=True))
    s_row = jnp.concatenate(pieces, axis=1)

    rank_chunks = []
    for c in range(T // RC):
        sc = s_col[c * RC:(c + 1) * RC]
        iid = lax.broadcasted_iota(jnp.int32, (RC, 1), 0) + c * RC
        jid = lax.broadcasted_iota(jnp.int32, (RC, T), 1)
        beats = (s_row > sc) | ((s_row == sc) & (jid < iid))
        rank_chunks.append(
            jnp.sum(beats.astype(jnp.float32), axis=1, keepdims=True))
    ranks = jnp.concatenate(rank_chunks, axis=0).astype(jnp.int32)

    base = pl.program_id(0) * T
    ival = lax.broadcasted_iota(jnp.int32, (T, RC), 0).astype(jnp.float32)
    for c in range(T // RC):
        rr = lax.broadcasted_iota(jnp.int32, (T, RC), 1) + c * RC
        onehot = ranks == rr
        vals = jnp.sum(jnp.where(onehot, ival, 0.0), axis=0, keepdims=True)
        perm_ref[:, :, c * RC:(c + 1) * RC] = vals.astype(jnp.int32)[None] + base


def _router(x, w_router):
    return pl.pallas_call(
        _router_body,
        grid=(B,),
        in_specs=[pl.BlockSpec((1, T, D), lambda b: (b, 0, 0)),
                  pl.BlockSpec((D, 1), lambda b: (0, 0))],
        out_specs=pl.BlockSpec((1, 1, T), lambda b: (b, 0, 0)),
        out_shape=jax.ShapeDtypeStruct((B, 1, T), jnp.int32),
    )(x, w_router).reshape(B, T)


_NW = 32
_RG = (B * CAP) // _NW


def _sel_offset(wid):
    b = wid // (CAP // _RG)
    r0 = wid * _RG - b * CAP
    return b * T + r0, b * T + CAP + r0


@functools.cache
def _sc_kernels():
    info = plsc.get_sparse_core_info()
    nc = info.num_cores
    assert nc * info.num_subcores == _NW
    mesh = plsc.VectorSubcoreMesh(core_axis_name="c", subcore_axis_name="s")

    @functools.partial(
        pl.kernel, mesh=mesh,
        out_type=jax.ShapeDtypeStruct((B * CAP, D), jnp.float32),
        scratch_types=[pltpu.VMEM((_RG,), jnp.int32),
                       pltpu.VMEM((_RG, D), jnp.float32),
                       pltpu.SemaphoreType.DMA])
    def sc_gather(x2_hbm, gidx_hbm, out_hbm, idx_v, rows_v, sem):
        wid = lax.axis_index("s") * nc + lax.axis_index("c")
        off_sel, _ = _sel_offset(wid)
        pltpu.sync_copy(gidx_hbm.at[pl.ds(off_sel, _RG)], idx_v)
        pltpu.async_copy(x2_hbm.at[idx_v], rows_v, sem).wait()
        pltpu.sync_copy(rows_v, out_hbm.at[pl.ds(wid * _RG, _RG)])

    @functools.partial(pl.kernel, mesh=mesh, out_type=())
    def sc_init_copy(x2_hbm, out_hbm):
        wid = lax.axis_index("s") * nc + lax.axis_index("c")
        base = wid * ((B * T) // _NW)
        pltpu.sync_copy(x2_hbm.at[pl.ds(base, (B * T) // _NW)],
                        out_hbm.at[pl.ds(base, (B * T) // _NW)])

    @functools.partial(
        pl.kernel, mesh=mesh, out_type=(),
        scratch_types=[pltpu.VMEM((_RG,), jnp.int32),
                       pltpu.VMEM((_RG, D), jnp.float32),
                       pltpu.SemaphoreType.DMA])
    def sc_scatter(proc_hbm, gidx_hbm, out_hbm, idx_v, rows_v, sem):
        wid = lax.axis_index("s") * nc + lax.axis_index("c")
        off_sel, _ = _sel_offset(wid)
        pltpu.sync_copy(gidx_hbm.at[pl.ds(off_sel, _RG)], idx_v)
        pltpu.sync_copy(proc_hbm.at[pl.ds(wid * _RG, _RG)], rows_v)
        pltpu.async_copy(rows_v, out_hbm.at[idx_v], sem).wait()

    return sc_gather, sc_scatter, sc_init_copy


_FC = 1024


def _mlp_body(sel_ref, w1_ref, b1_ref, w2_ref, b2_ref, out_ref):
    k = pl.program_id(0)
    h = lax.dot_general(sel_ref[...], w1_ref[...], (((1,), (0,)), ((), ())),
                        preferred_element_type=jnp.float32)
    h = jax.nn.gelu(h + b1_ref[...])
    o = lax.dot_general(h, w2_ref[...], (((1,), (0,)), ((), ())),
                        preferred_element_type=jnp.float32)

    @pl.when(k == 0)
    def _():
        out_ref[...] = o + b2_ref[...]

    @pl.when(k != 0)
    def _():
        out_ref[...] += o


def _mlp(selected, w1b, b1r, w2b, b2r):
    n = B * CAP
    return pl.pallas_call(
        _mlp_body,
        grid=(FF // _FC,),
        in_specs=[pl.BlockSpec((n, D), lambda k: (0, 0)),
                  pl.BlockSpec((D, _FC), lambda k: (0, k)),
                  pl.BlockSpec((1, _FC), lambda k: (0, k)),
                  pl.BlockSpec((_FC, D), lambda k: (k, 0)),
                  pl.BlockSpec((1, D), lambda k: (0, 0))],
        out_specs=pl.BlockSpec((n, D), lambda k: (0, 0)),
        out_shape=jax.ShapeDtypeStruct((n, D), jnp.float32),
        compiler_params=pltpu.CompilerParams(
            dimension_semantics=("arbitrary",)),
    )(selected, w1b, b1r, w2b, b2r)


def kernel(x, W_router, W1, b1, W2, b2):
    sc_gather, sc_scatter, sc_init_copy = _sc_kernels()
    x2 = x.reshape(B * T, D)
    out_ref = jax.new_ref(lax.empty((B * T, D), jnp.float32))
    sc_init_copy(x2, out_ref)
    glob_perm = _router(x, W_router)
    gidx = glob_perm.reshape(B * T)
    selected = sc_gather(x2, gidx)
    processed = _mlp(selected, W1, b1.reshape(1, FF), W2, b2.reshape(1, D))
    sc_scatter(processed, gidx, out_ref)
    return jax.freeze(out_ref).reshape(B, T, D)


# speedup vs baseline: 5.5539x; 5.5539x over previous
import functools

import jax
import jax.numpy as jnp
from jax import lax
from jax.experimental import pallas as pl
from jax.experimental.pallas import tpu as pltpu
from jax.experimental.pallas import tpu_sc as plsc

B, T, D, FF = 2, 2048, 1024, 4096
CAP = T // 2
RC = 256


def _router_body(x_ref, w_ref, perm_ref):
    xb = x_ref[0]
    w = w_ref[...]
    s_col = lax.dot_general(
        xb.astype(jnp.bfloat16), w.astype(jnp.bfloat16),
        (((1,), (0,)), ((), ())),
        preferred_element_type=jnp.float32)

    pieces = []
    for c in range(T // RC):
        ii = lax.broadcasted_iota(jnp.int32, (T, RC), 0)
        jj = lax.broadcasted_iota(jnp.int32, (T, RC), 1)
        sel = ii == (jj + c * RC)
        pieces.append(jnp.max(jnp.where(sel, s_col, -jnp.inf), axis=0,
                              keepdims=True))
    s_row = jnp.concatenate(pieces, axis=1)

    rank_chunks = []
    for c in range(T // RC):
        sc = s_col[c * RC:(c + 1) * RC]
        iid = lax.broadcasted_iota(jnp.int32, (RC, 1), 0) + c * RC
        jid = lax.broadcasted_iota(jnp.int32, (RC, T), 1)
        beats = (s_row > sc) | ((s_row == sc) & (jid < iid))
        rank_chunks.append(
            jnp.sum(beats.astype(jnp.float32), axis=1, keepdims=True))
    ranks = jnp.concatenate(rank_chunks, axis=0).astype(jnp.int32)

    base = pl.program_id(0) * T
    ival = lax.broadcasted_iota(jnp.int32, (T, RC), 0).astype(jnp.float32)
    for c in range(T // RC):
        rr = lax.broadcasted_iota(jnp.int32, (T, RC), 1) + c * RC
        onehot = ranks == rr
        vals = jnp.sum(jnp.where(onehot, ival, 0.0), axis=0, keepdims=True)
        perm_ref[:, :, c * RC:(c + 1) * RC] = vals.astype(jnp.int32)[None] + base


def _router(x, w_router):
    return pl.pallas_call(
        _router_body,
        grid=(B,),
        in_specs=[pl.BlockSpec((1, T, D), lambda b: (b, 0, 0)),
                  pl.BlockSpec((D, 1), lambda b: (0, 0))],
        out_specs=pl.BlockSpec((1, 1, T), lambda b: (b, 0, 0)),
        out_shape=jax.ShapeDtypeStruct((B, 1, T), jnp.int32),
    )(x, w_router).reshape(B, T)


_NW = 32
_RG = (B * CAP) // _NW


def _sel_offset(wid):
    b = wid // (CAP // _RG)
    r0 = wid * _RG - b * CAP
    return b * T + r0, b * T + CAP + r0


@functools.cache
def _sc_kernels():
    info = plsc.get_sparse_core_info()
    nc = info.num_cores
    assert nc * info.num_subcores == _NW
    mesh = plsc.VectorSubcoreMesh(core_axis_name="c", subcore_axis_name="s")

    @functools.partial(
        pl.kernel, mesh=mesh,
        out_type=jax.ShapeDtypeStruct((B * CAP, D), jnp.float32),
        scratch_types=[pltpu.VMEM((_RG,), jnp.int32),
                       pltpu.VMEM((_RG, D), jnp.float32),
                       pltpu.SemaphoreType.DMA])
    def sc_gather(x2_hbm, gidx_hbm, out_hbm, idx_v, rows_v, sem):
        wid = lax.axis_index("s") * nc + lax.axis_index("c")
        off_sel, _ = _sel_offset(wid)
        pltpu.sync_copy(gidx_hbm.at[pl.ds(off_sel, _RG)], idx_v)
        pltpu.async_copy(x2_hbm.at[idx_v], rows_v, sem).wait()
        pltpu.sync_copy(rows_v, out_hbm.at[pl.ds(wid * _RG, _RG)])

    @functools.partial(
        pl.kernel, mesh=mesh, out_type=(),
        scratch_types=[pltpu.VMEM((_RG, D), jnp.float32),
                       pltpu.VMEM((_RG, D), jnp.float32)])
    def sc_init_copy(x2_hbm, out_hbm, buf_a, buf_b):
        wid = lax.axis_index("s") * nc + lax.axis_index("c")
        base = wid * ((B * T) // _NW)
        for c, buf in ((0, buf_a), (1, buf_b)):
            pltpu.sync_copy(x2_hbm.at[pl.ds(base + c * _RG, _RG)], buf)
            pltpu.sync_copy(buf, out_hbm.at[pl.ds(base + c * _RG, _RG)])

    @functools.partial(
        pl.kernel, mesh=mesh, out_type=(),
        scratch_types=[pltpu.VMEM((_RG,), jnp.int32),
                       pltpu.VMEM((_RG, D), jnp.float32),
                       pltpu.SemaphoreType.DMA])
    def sc_scatter(proc_hbm, gidx_hbm, out_hbm, idx_v, rows_v, sem):
        wid = lax.axis_index("s") * nc + lax.axis_index("c")
        off_sel, _ = _sel_offset(wid)
        pltpu.sync_copy(gidx_hbm.at[pl.ds(off_sel, _RG)], idx_v)
        pltpu.sync_copy(proc_hbm.at[pl.ds(wid * _RG, _RG)], rows_v)
        pltpu.async_copy(rows_v, out_hbm.at[idx_v], sem).wait()

    return sc_gather, sc_scatter, sc_init_copy


_FC = 1024


def _mlp_body(sel_ref, w1_ref, b1_ref, w2_ref, b2_ref, out_ref):
    k = pl.program_id(0)
    h = lax.dot_general(sel_ref[...], w1_ref[...], (((1,), (0,)), ((), ())),
                        preferred_element_type=jnp.float32)
    h = jax.nn.gelu(h + b1_ref[...])
    o = lax.dot_general(h, w2_ref[...], (((1,), (0,)), ((), ())),
                        preferred_element_type=jnp.float32)

    @pl.when(k == 0)
    def _():
        out_ref[...] = o + b2_ref[...]

    @pl.when(k != 0)
    def _():
        out_ref[...] += o


def _mlp(selected, w1b, b1r, w2b, b2r):
    n = B * CAP
    return pl.pallas_call(
        _mlp_body,
        grid=(FF // _FC,),
        in_specs=[pl.BlockSpec((n, D), lambda k: (0, 0)),
                  pl.BlockSpec((D, _FC), lambda k: (0, k)),
                  pl.BlockSpec((1, _FC), lambda k: (0, k)),
                  pl.BlockSpec((_FC, D), lambda k: (k, 0)),
                  pl.BlockSpec((1, D), lambda k: (0, 0))],
        out_specs=pl.BlockSpec((n, D), lambda k: (0, 0)),
        out_shape=jax.ShapeDtypeStruct((n, D), jnp.float32),
        compiler_params=pltpu.CompilerParams(
            dimension_semantics=("arbitrary",)),
    )(selected, w1b, b1r, w2b, b2r)


def kernel(x, W_router, W1, b1, W2, b2):
    sc_gather, sc_scatter, sc_init_copy = _sc_kernels()
    x2 = x.reshape(B * T, D)
    out_ref = jax.new_ref(lax.empty((B * T, D), jnp.float32))
    sc_init_copy(x2, out_ref)
    glob_perm = _router(x, W_router)
    gidx = glob_perm.reshape(B * T)
    selected = sc_gather(x2, gidx)
    processed = _mlp(selected, W1, b1.reshape(1, FF), W2, b2.reshape(1, D))
    sc_scatter(processed, gidx, out_ref)
    return jax.freeze(out_ref).reshape(B, T, D)
